# baseline (speedup 1.0000x reference)
import functools

import jax
import jax.numpy as jnp
from jax import lax
from jax.experimental import pallas as pl
from jax.experimental.pallas import tpu as pltpu
from jax.experimental.pallas import tpu_sc as plsc

_NUM_CORES = 2
_NUM_SUBCORES = 16
_NUM_WORKERS = _NUM_CORES * _NUM_SUBCORES
_KT = 8
_CLS = 128
_SH = 16
_PADW = 192


def _make_body(L1, L2, D, V, q_rows, cls_per_w, win_cols):
    maxp = (V - 1) // 2
    band_lo = L1 - 1 - maxp
    n_fill = win_cols // _SH
    n_band = (2 * maxp) // _SH + 2
    groups = D // _KT
    passes = [
        (g, d) for g in range(groups) for d in range(cls_per_w)
    ]

    def body(embsh_hbm, out_hbm, tabs, slab0, slab1, slab2, sem):
        w = lax.axis_index("c") * _NUM_SUBCORES + lax.axis_index("s")
        slabs = (slab0, slab1, slab2)

        def cls_of(d):
            return cls_per_w * w + d

        def build(p):
            g, d = passes[p]
            slab = slabs[p % 3]
            cls = cls_of(d)
            colbase = L1 - 1 - (cls + _CLS * (q_rows - 1))
            u_lo = band_lo - colbase
            hc = u_lo // _SH
            sidx = u_lo - hc * _SH
            for kk in range(_KT):
                v0 = tabs[sidx, kk, pl.ds(0, _SH)]
                v1 = tabs[sidx, kk, pl.ds(_PADW - _SH, _SH)]

                def fill0(uc, c, slab=slab, kk=kk, v0=v0):
                    slab[kk, pl.ds(uc * _SH, _SH)] = v0
                    return c

                def fill1(uc, c, slab=slab, kk=kk, v1=v1):
                    slab[kk, pl.ds(uc * _SH, _SH)] = v1
                    return c

                def band(bc, c, slab=slab, kk=kk):
                    slab[kk, pl.ds((hc + bc) * _SH, _SH)] = tabs[
                        sidx, kk, pl.ds((bc + 1) * _SH, _SH)
                    ]
                    return c

                lax.fori_loop(0, hc, fill0, 0)
                lax.fori_loop(hc + n_band, n_fill, fill1, 0)
                lax.fori_loop(0, n_band, band, 0)

        def desc(p, q):
            g, d = passes[p]
            slab = slabs[p % 3]
            i = cls_of(d) + _CLS * q
            return pltpu.make_async_copy(
                slab.at[:, pl.ds((q_rows - 1 - q) * _CLS, L2)],
                out_hbm.at[i].at[pl.ds(g * _KT, _KT), :],
                sem,
            )

        def fire(p):
            def go(q, c):
                desc(p, q).start()
                return c

            lax.fori_loop(0, q_rows, go, 0)

        def drain(p):
            def go(q, c):
                desc(p, q).wait()
                return c

            lax.fori_loop(0, q_rows, go, 0)

        n = len(passes)
        for p in range(n):
            if passes[p][1] == 0:
                pltpu.sync_copy(embsh_hbm.at[passes[p][0]], tabs)
            build(p)
            if p > 1:
                drain(p - 2)
            fire(p)
        drain(n - 2)
        drain(n - 1)

    return body


@functools.lru_cache(maxsize=None)
def _make_kernel(L1, L2, D, V):
    assert L1 % _CLS == 0 and L2 % _CLS == 0 and D % _KT == 0
    q_rows = L1 // _CLS
    cls_per_w = _CLS // _NUM_WORKERS
    win_cols = L2 + _CLS * (q_rows - 1)

    body = _make_body(L1, L2, D, V, q_rows, cls_per_w, win_cols)
    sc_call = pl.kernel(
        body,
        out_type=jax.ShapeDtypeStruct((L1, D, L2), jnp.float32),
        mesh=plsc.VectorSubcoreMesh(core_axis_name="c", subcore_axis_name="s"),
        scratch_types=[
            pltpu.VMEM((_SH, _KT, _PADW), jnp.float32),
            pltpu.VMEM((_KT, win_cols), jnp.float32),
            pltpu.VMEM((_KT, win_cols), jnp.float32),
            pltpu.VMEM((_KT, win_cols), jnp.float32),
            pltpu.SemaphoreType.DMA,
        ],
    )

    @jax.jit
    def run(embeddings):
        embt = embeddings.T
        mats = []
        for s in range(_SH):
            left = jnp.repeat(embt[:, :1], _SH + s, axis=1)
            right = jnp.repeat(embt[:, -1:], _PADW - _SH - s - V, axis=1)
            mats.append(jnp.concatenate([left, embt, right], axis=1))
        embsh = jnp.stack(mats)
        groups = embeddings.shape[1] // _KT
        embsh = embsh.reshape(_SH, groups, _KT, _PADW).transpose(1, 0, 2, 3)
        out_ikj = sc_call(embsh)
        return jnp.transpose(out_ikj, (0, 2, 1))

    return run


def kernel(seq1, seq2, embeddings):
    L1 = seq1.shape[1]
    L2 = seq2.shape[1]
    V, D = embeddings.shape
    return _make_kernel(L1, L2, D, V)(embeddings)


# speedup vs baseline: 1.1155x; 1.1155x over previous
import functools

import jax
import jax.numpy as jnp
from jax import lax
from jax.experimental import pallas as pl
from jax.experimental.pallas import tpu as pltpu
from jax.experimental.pallas import tpu_sc as plsc

_NUM_CORES = 2
_NUM_SUBCORES = 16
_NUM_WORKERS = _NUM_CORES * _NUM_SUBCORES
_KT = 8
_CLS = 128
_SH = 16
_PADW = 192


def _make_body(L1, L2, D, V, q_rows, cls_per_w, k_half, win_cols):
    maxp = (V - 1) // 2
    band_lo = L1 - 1 - maxp
    n_fill = win_cols // _SH
    n_band = (2 * maxp) // _SH + 2
    halves = D // k_half
    quarters = k_half // _KT
    passes = [
        (half, qt, d)
        for half in range(halves)
        for qt in range(quarters)
        for d in range(cls_per_w)
    ]

    def body(embsh_hbm, out_hbm, tabs, slab0, slab1, sem):
        w = lax.axis_index("c") * _NUM_SUBCORES + lax.axis_index("s")
        slabs = (slab0, slab1)

        def cls_of(d):
            return cls_per_w * w + d

        def build(p):
            half, qt, d = passes[p]
            slab = slabs[p % 2]
            cls = cls_of(d)
            colbase = L1 - 1 - (cls + _CLS * (q_rows - 1))
            u_lo = band_lo - colbase
            hc = u_lo // _SH
            sidx = u_lo - hc * _SH
            for kk in range(_KT):
                kh = qt * _KT + kk
                v0 = tabs[sidx, kh, pl.ds(0, _SH)]
                v1 = tabs[sidx, kh, pl.ds(_PADW - _SH, _SH)]

                def fill0(uc, c, slab=slab, kk=kk, v0=v0):
                    slab[kk, pl.ds(uc * _SH, _SH)] = v0
                    return c

                def fill1(uc, c, slab=slab, kk=kk, v1=v1):
                    slab[kk, pl.ds(uc * _SH, _SH)] = v1
                    return c

                def band(bc, c, slab=slab, kk=kk, kh=kh):
                    slab[kk, pl.ds((hc + bc) * _SH, _SH)] = tabs[
                        sidx, kh, pl.ds((bc + 1) * _SH, _SH)
                    ]
                    return c

                lax.fori_loop(0, hc, fill0, 0)
                lax.fori_loop(hc + n_band, n_fill, fill1, 0)
                lax.fori_loop(0, n_band, band, 0)

        def desc(p, q):
            half, qt, d = passes[p]
            slab = slabs[p % 2]
            i = cls_of(d) + _CLS * q
            return pltpu.make_async_copy(
                slab.at[:, pl.ds((q_rows - 1 - q) * _CLS, L2)],
                out_hbm.at[i].at[pl.ds((half * quarters + qt) * _KT, _KT), :],
                sem,
            )

        def fire(p):
            def go(q, c):
                desc(p, q).start()
                return c

            lax.fori_loop(0, q_rows, go, 0)

        def drain(p):
            def go(q, c):
                desc(p, q).wait()
                return c

            lax.fori_loop(0, q_rows, go, 0)

        n = len(passes)
        for p in range(n):
            if passes[p][1] == 0 and passes[p][2] == 0:
                pltpu.sync_copy(embsh_hbm.at[passes[p][0]], tabs)
            build(p)
            if p > 0:
                drain(p - 1)
            fire(p)
        drain(n - 1)

    return body


@functools.lru_cache(maxsize=None)
def _make_kernel(L1, L2, D, V):
    assert L1 % _CLS == 0 and L2 % _CLS == 0 and D % _KT == 0
    q_rows = L1 // _CLS
    cls_per_w = _CLS // _NUM_WORKERS
    k_half = 16 if D > 16 else D
    win_cols = L2 + _CLS * (q_rows - 1)

    body = _make_body(L1, L2, D, V, q_rows, cls_per_w, k_half, win_cols)
    sc_call = pl.kernel(
        body,
        out_type=jax.ShapeDtypeStruct((L1, D, L2), jnp.float32),
        mesh=plsc.VectorSubcoreMesh(core_axis_name="c", subcore_axis_name="s"),
        scratch_types=[
            pltpu.VMEM((_SH, k_half, _PADW), jnp.float32),
            pltpu.VMEM((_KT, win_cols), jnp.float32),
            pltpu.VMEM((_KT, win_cols), jnp.float32),
            pltpu.SemaphoreType.DMA,
        ],
    )

    @jax.jit
    def run(embeddings):
        embt = embeddings.T
        mats = []
        for s in range(_SH):
            left = jnp.repeat(embt[:, :1], _SH + s, axis=1)
            right = jnp.repeat(embt[:, -1:], _PADW - _SH - s - V, axis=1)
            mats.append(jnp.concatenate([left, embt, right], axis=1))
        embsh = jnp.stack(mats)
        k_half = 16 if embeddings.shape[1] > 16 else embeddings.shape[1]
        halves = embeddings.shape[1] // k_half
        embsh = embsh.reshape(_SH, halves, k_half, _PADW).transpose(1, 0, 2, 3)
        out_ikj = sc_call(embsh)
        return jnp.transpose(out_ikj, (0, 2, 1))

    return run


def kernel(seq1, seq2, embeddings):
    L1 = seq1.shape[1]
    L2 = seq2.shape[1]
    V, D = embeddings.shape
    return _make_kernel(L1, L2, D, V)(embeddings)
